# baseline (speedup 1.0000x reference)
import functools

import jax
import jax.numpy as jnp
from jax import lax
from jax.experimental import pallas as pl
from jax.experimental.pallas import tpu as pltpu
from jax.experimental.pallas import tpu_sc as plsc

SH_V, YU_V = 24, 40
SH_D, YU_D = 32, 32
OUT_D = SH_D + YU_D
TAB_ROWS = SH_V * YU_V
NC, NS, L = 2, 16, 16
NW = NC * NS
GBLK = 128


def _prelude_body(sh_ref, yu_ref, s_ref, y_ref, tab_ref, comb_ref):
    sh = sh_ref[...]
    yu = yu_ref[...]
    shb = jnp.broadcast_to(sh[:, None, :], (SH_V, YU_V, SH_D)).reshape(
        TAB_ROWS, SH_D)
    yub = jnp.broadcast_to(yu[None, :, :], (SH_V, YU_V, YU_D)).reshape(
        TAB_ROWS, YU_D)
    tab_ref[...] = jnp.concatenate([shb, yub], axis=-1)
    comb = s_ref[...] * YU_V + y_ref[...]
    seq = comb.shape[1]
    comb_ref[...] = jnp.stack(
        [comb[:, :GBLK], comb[:, seq - GBLK:]], axis=1)


def _prelude(sh_table, yu_table, s_idx, y_idx):
    return pl.pallas_call(
        _prelude_body,
        out_shape=(
            jax.ShapeDtypeStruct((TAB_ROWS, OUT_D), jnp.float32),
            jax.ShapeDtypeStruct((s_idx.shape[0], 2, GBLK), jnp.int32),
        ),
    )(sh_table, yu_table, s_idx, y_idx)


def _make_sc_kernel(batch, seq):
    rows_per_w = batch // NW
    off2 = seq - GBLK
    mesh = plsc.VectorSubcoreMesh(
        core_axis_name="c", subcore_axis_name="s",
        num_cores=NC, num_subcores=NS)

    @functools.partial(
        pl.kernel,
        out_type=jax.ShapeDtypeStruct((batch, seq, OUT_D), jnp.float32),
        mesh=mesh,
        scratch_types=[
            pltpu.VMEM((rows_per_w, 2, GBLK), jnp.int32),
            [pltpu.VMEM((seq, OUT_D), jnp.float32) for _ in range(2)],
            [pltpu.SemaphoreType.DMA for _ in range(2)],
            [pltpu.SemaphoreType.DMA for _ in range(2)],
            pltpu.SemaphoreType.DMA,
            pltpu.VMEM_SHARED((TAB_ROWS, OUT_D), jnp.float32),
        ],
    )
    def sc_kernel(comb_hbm, table_hbm, out_hbm,
                  comb_v, bufs, gsems, wsems, lsem, table_sh):
        sid = lax.axis_index("s")
        wid = sid * NC + lax.axis_index("c")

        @pl.when(sid == 0)
        def _stage_table():
            pltpu.sync_copy(table_hbm, table_sh)

        pltpu.sync_copy(comb_hbm.at[pl.ds(wid * rows_per_w, rows_per_w)],
                        comb_v)
        plsc.subcore_barrier()

        def gathers(b_local, slot):
            pltpu.async_copy(
                table_sh.at[comb_v.at[b_local, 0]],
                bufs[slot].at[pl.ds(0, GBLK)], gsems[slot])
            pltpu.async_copy(
                table_sh.at[comb_v.at[b_local, 1]],
                bufs[slot].at[pl.ds(off2, GBLK)], gsems[slot])

        def drain_gathers(b_local, slot):
            pltpu.make_async_copy(
                table_sh.at[comb_v.at[b_local, 0]],
                bufs[slot].at[pl.ds(0, GBLK)], gsems[slot]).wait()
            pltpu.make_async_copy(
                table_sh.at[comb_v.at[b_local, 1]],
                bufs[slot].at[pl.ds(off2, GBLK)], gsems[slot]).wait()

        gathers(0, 0)

        def step(b, _):
            for r in range(2):
                @pl.when(b % 2 == r)
                def _do():
                    row = wid * rows_per_w + b
                    drain_gathers(b, r)
                    pltpu.async_copy(bufs[r], out_hbm.at[row], wsems[r])

                    @pl.when(b + 1 < rows_per_w)
                    def _prefetch():
                        @pl.when(b >= 1)
                        def _reclaim():
                            prow = wid * rows_per_w + b - 1
                            pltpu.make_async_copy(
                                bufs[1 - r], out_hbm.at[prow],
                                wsems[1 - r]).wait()
                        gathers(b + 1, 1 - r)
            return 0

        lax.fori_loop(0, rows_per_w, step, 0)

        for b in (rows_per_w - 2, rows_per_w - 1):
            r = b % 2
            row = wid * rows_per_w + b
            pltpu.make_async_copy(bufs[r], out_hbm.at[row], wsems[r]).wait()

    return sc_kernel


def kernel(shengmu_indices, yunmu_indices, shengmu_table, yunmu_table):
    batch, seq = shengmu_indices.shape
    assert batch % NW == 0
    assert seq > GBLK // 2 and (seq - GBLK) % 8 == 0 and seq % 8 == 0

    table, comb = _prelude(shengmu_table, yunmu_table,
                           shengmu_indices, yunmu_indices)
    return _make_sc_kernel(batch, seq)(comb, table)
